# baseline (speedup 1.0000x reference)
import functools

import jax
import jax.numpy as jnp
from jax import lax
from jax.experimental import pallas as pl
from jax.experimental.pallas import tpu as pltpu
from jax.experimental.pallas import tpu_sc as plsc

D = 384
P = 512
H = 1024
NC = 2
NS = 16
NWJ = 16
NWI = 2
JW = D // NWJ
IW = D // NWI
CH = 48
NCH = IW // CH
NB = 3
L = 16


def _body(table_hbm, out_hbm, jtab, itab, outbuf, sem0, sem1, sem2):
    wid = lax.axis_index("s") * NC + lax.axis_index("c")
    jbase = pl.multiple_of((wid % NWJ) * JW, 8)
    ibase = pl.multiple_of((wid // NWJ) * IW, 8)
    sems = (sem0, sem1, sem2)

    pltpu.sync_copy(table_hbm.at[pl.ds(jbase, JW)], jtab)

    @pl.loop(0, JW)
    def _init_j(jj):
        @pl.loop(0, P // L, unroll=4)
        def _(k):
            v = jtab[jj, pl.ds(k * L, L)]
            for b in range(NB):
                outbuf[b, jj, pl.ds(P + k * L, L)] = v

    def fill_i(b, irow):
        @pl.loop(0, P // L, unroll=4)
        def _(k):
            v = itab[irow, pl.ds(k * L, L)]
            for jj in range(JW):
                outbuf[b, jj, pl.ds(k * L, L)] = v

    def start_out(b, i):
        rowstart = pl.multiple_of(i * D + jbase, 8)
        pltpu.async_copy(
            outbuf.at[b], out_hbm.at[pl.ds(rowstart, JW)], sems[b])

    def wait_out(b):
        pltpu.make_async_copy(
            outbuf.at[b], out_hbm.at[pl.ds(jbase, JW)], sems[b]).wait()

    for c in range(NCH):
        cbase = pl.multiple_of(ibase + c * CH, 8)
        pltpu.sync_copy(table_hbm.at[pl.ds(cbase, CH)], itab)
        if c == 0:
            for b in range(NB):
                fill_i(b, b)
                start_out(b, ibase + b)
            lo = NB
        else:
            lo = 0

        @pl.loop(lo, CH, step=NB)
        def _main(ii):
            i = ibase + c * CH + ii
            for b in range(NB):
                wait_out(b)
                fill_i(b, ii + b)
                start_out(b, i + b)

    for b in range(NB):
        wait_out(b)


@jax.jit
def _positions(table):
    mesh = plsc.VectorSubcoreMesh(
        core_axis_name="c", subcore_axis_name="s",
        num_cores=NC, num_subcores=NS)
    f = pl.kernel(
        _body,
        out_type=jax.ShapeDtypeStruct((D * D, H), jnp.float32),
        mesh=mesh,
        scratch_types=[
            pltpu.VMEM((JW, P), jnp.float32),
            pltpu.VMEM((CH, P), jnp.float32),
            pltpu.VMEM((NB, JW, H), jnp.float32),
            pltpu.SemaphoreType.DMA,
            pltpu.SemaphoreType.DMA,
            pltpu.SemaphoreType.DMA,
        ],
    )
    return f(table)


def kernel(table, spatial_shape):
    out2d = _positions(table)
    return out2d.reshape(D, D, H)


# speedup vs baseline: 1.0968x; 1.0968x over previous
import jax
import jax.numpy as jnp
from jax import lax
from jax.experimental import pallas as pl
from jax.experimental.pallas import tpu as pltpu
from jax.experimental.pallas import tpu_sc as plsc

D = 384
P = 512
H = 1024
NC = 2
NS = 16
SW = 24
R = 8
NG = D // R


def _body(table_hbm, out_hbm, buf, sem):
    wid = lax.axis_index("s") * NC + lax.axis_index("c")
    is_i_writer = wid < 16
    base = pl.multiple_of((wid % 16) * SW, 8)

    @pl.when(is_i_writer)
    def _():
        for r in range(R):
            pltpu.sync_copy(table_hbm.at[pl.ds(base, SW)],
                            buf.at[:, r, :])

    @pl.when(jnp.logical_not(is_i_writer))
    def _():
        bj = buf.reshape(R, SW, P)
        for r in range(R):
            pltpu.sync_copy(table_hbm.at[pl.ds(base, SW)], bj.at[r])

    @pl.when(is_i_writer)
    def _():
        @pl.loop(0, NG)
        def _(g):
            g8 = pl.multiple_of(g * R, 8)
            pltpu.async_copy(
                buf,
                out_hbm.at[pl.ds(base, SW), pl.ds(g8, R), pl.ds(0, P)],
                sem)

        @pl.loop(0, NG)
        def _(g):
            pltpu.make_async_copy(
                buf,
                out_hbm.at[pl.ds(base, SW), pl.ds(0, R), pl.ds(0, P)],
                sem).wait()

    @pl.when(jnp.logical_not(is_i_writer))
    def _():
        bj = buf.reshape(R, SW, P)

        @pl.loop(0, NG)
        def _(g):
            g8 = pl.multiple_of(g * R, 8)
            pltpu.async_copy(
                bj,
                out_hbm.at[pl.ds(g8, R), pl.ds(base, SW), pl.ds(P, P)],
                sem)

        @pl.loop(0, NG)
        def _(g):
            pltpu.make_async_copy(
                bj,
                out_hbm.at[pl.ds(0, R), pl.ds(base, SW), pl.ds(P, P)],
                sem).wait()


@jax.jit
def _positions(table):
    mesh = plsc.VectorSubcoreMesh(
        core_axis_name="c", subcore_axis_name="s",
        num_cores=NC, num_subcores=NS)
    f = pl.kernel(
        _body,
        out_type=jax.ShapeDtypeStruct((D, D, H), jnp.float32),
        mesh=mesh,
        scratch_types=[
            pltpu.VMEM((SW, R, P), jnp.float32),
            pltpu.SemaphoreType.DMA,
        ],
    )
    return f(table)


def kernel(table, spatial_shape):
    return _positions(table)
